# baseline (speedup 1.0000x reference)
import functools
import math

import numpy as np
import jax
import jax.numpy as jnp
from jax import lax
from jax.experimental import pallas as pl
from jax.experimental.pallas import tpu as pltpu
from jax.experimental.pallas import tpu_sc as plsc

N_SAMPLES = 8192
NUM_BINS = 8192
NB = NUM_BINS - 1
NROWS = 50
BS = 8
NSTEPS = N_SAMPLES // BS
R64, L128 = 64, 128

K_CAT = (1832780943, 270669613)
K_UNIF = (64467757, 2916123636)
K_NORM = (2465931498, 255383827)

TINY = np.float32(np.finfo(np.float32).tiny)
NORM_LO = np.float32(np.nextafter(np.float32(-1.0), np.float32(0.0)))
NORM_SCALE = np.float32(np.float32(1.0) - NORM_LO)
SQRT2 = np.float32(np.sqrt(2.0))
PI32 = np.float32(math.pi)


def _c32(x):
    return jnp.asarray(np.uint32(x).view(np.int32), dtype=jnp.int32)


def _threefry_xor(key, j):
    k1, k2 = key
    ks0 = _c32(k1)
    ks1 = _c32(k2)
    ks2 = _c32(np.uint32(k1) ^ np.uint32(k2) ^ np.uint32(0x1BD11BDA))

    def rotl(v, d):
        return jnp.bitwise_or(
            jnp.left_shift(v, d), lax.shift_right_logical(v, 32 - d))

    x0 = jnp.full_like(j, 0) + ks0
    x1 = j + ks1
    ks = (ks0, ks1, ks2)
    rot_a = (13, 15, 26, 6)
    rot_b = (17, 29, 16, 24)
    for r in range(5):
        for d in (rot_a if r % 2 == 0 else rot_b):
            x0 = x0 + x1
            x1 = rotl(x1, d)
            x1 = jnp.bitwise_xor(x1, x0)
        x0 = x0 + ks[(r + 1) % 3]
        x1 = x1 + ks[(r + 2) % 3] + (r + 1)
    return jnp.bitwise_xor(x0, x1)


def _bits_to_unit_float(bits):
    mant = jnp.bitwise_or(lax.shift_right_logical(bits, 9), _c32(0x3F800000))
    return lax.bitcast_convert_type(mant, jnp.float32) - jnp.float32(1.0)


def _erf_inv(x):
    w = -jnp.log((jnp.float32(1.0) - x) * (jnp.float32(1.0) + x))
    ws = w - jnp.float32(2.5)
    p_s = jnp.float32(2.81022636e-08)
    for c in (3.43273939e-07, -3.5233877e-06, -4.39150654e-06, 0.00021858087,
              -0.00125372503, -0.00417768164, 0.246640727, 1.50140941):
        p_s = jnp.float32(c) + p_s * ws
    wb = jnp.sqrt(w) - jnp.float32(3.0)
    p_b = jnp.float32(-0.000200214257)
    for c in (0.000100950558, 0.00134934322, -0.00367342844, 0.00573950773,
              -0.0076224613, 0.00943887047, 1.00167406, 2.83297682):
        p_b = jnp.float32(c) + p_b * wb
    return jnp.where(w < jnp.float32(5.0), p_s, p_b) * x


def _tc_body(idx_ref, y_ref, bin_ref, u_ref, n_ref, logit_scr):
    pid = pl.program_id(0)
    r_i = lax.broadcasted_iota(jnp.int32, (R64, L128), 0)
    l_i = lax.broadcasted_iota(jnp.int32, (R64, L128), 1)
    b2 = r_i * L128 + l_i

    @pl.when(pid == 0)
    def _init():
        is_pad = jnp.logical_and(r_i == R64 - 1, l_i == L128 - 1)
        for d in range(NROWS):
            lg = jnp.log(jnp.maximum(y_ref[d], jnp.float32(1e-38)))
            logit_scr[d] = jnp.where(is_pad, jnp.float32(-1e30), lg)
        u_ref[...] = _bits_to_unit_float(_threefry_xor(K_UNIF, b2))
        fn = _bits_to_unit_float(_threefry_xor(K_NORM, b2))
        un = jnp.maximum(NORM_LO, fn * NORM_SCALE + NORM_LO)
        n_ref[...] = SQRT2 * _erf_inv(un)

    base = pid * BS
    for s in range(BS):
        i = base + s
        row = idx_ref[i]
        lrow = logit_scr[row]
        j = i * NB + b2
        f = _bits_to_unit_float(_threefry_xor(K_CAT, j))
        u = jnp.maximum(TINY, f + TINY)
        g = -jnp.log(-jnp.log(u))
        val = lrow + g
        m = jnp.max(val)
        cand = jnp.where(val == m, b2, jnp.int32(2**30))
        bin_ref[0, 0, s] = jnp.min(cand)


def _tc_call(std_idx, Y3, interpret=False):
    grid_spec = pltpu.PrefetchScalarGridSpec(
        num_scalar_prefetch=1,
        grid=(NSTEPS,),
        in_specs=[
            pl.BlockSpec((NROWS, R64, L128), lambda i, *_: (0, 0, 0)),
        ],
        out_specs=[
            pl.BlockSpec((1, 1, BS), lambda i, *_: (i, 0, 0),
                         memory_space=pltpu.SMEM),
            pl.BlockSpec((R64, L128), lambda i, *_: (0, 0)),
            pl.BlockSpec((R64, L128), lambda i, *_: (0, 0)),
        ],
        scratch_shapes=[pltpu.VMEM((NROWS, R64, L128), jnp.float32)],
    )
    return pl.pallas_call(
        _tc_body,
        grid_spec=grid_spec,
        out_shape=[
            jax.ShapeDtypeStruct((NSTEPS, 1, BS), jnp.int32),
            jax.ShapeDtypeStruct((R64, L128), jnp.float32),
            jax.ShapeDtypeStruct((R64, L128), jnp.float32),
        ],
        interpret=interpret,
    )(std_idx, Y3)


_NC = 2
_NS = 16
_NW = _NC * _NS
_CHUNK = N_SAMPLES // _NW
_LANES = 16


def _sc_body(xrow_hbm, sd_hbm, bin_hbm, sidx_hbm, u_hbm, n_hbm, out_hbm,
             xrow_v, sd_v, bin_v, sidx_v, u_v, n_v, out_v):
    wid = lax.axis_index("s") * _NC + lax.axis_index("c")
    base = wid * _CHUNK
    pltpu.sync_copy(xrow_hbm, xrow_v)
    pltpu.sync_copy(sd_hbm, sd_v)
    pltpu.sync_copy(bin_hbm.at[pl.ds(base, _CHUNK)], bin_v)
    pltpu.sync_copy(sidx_hbm.at[pl.ds(base, _CHUNK)], sidx_v)
    pltpu.sync_copy(u_hbm.at[pl.ds(base, _CHUNK)], u_v)
    pltpu.sync_copy(n_hbm.at[pl.ds(base, _CHUNK)], n_v)
    for k in range(_CHUNK // _LANES):
        sl = pl.ds(k * _LANES, _LANES)
        bi = bin_v[sl]
        xs = plsc.load_gather(xrow_v, [bi])
        xe = plsc.load_gather(xrow_v, [bi + 1])
        st = plsc.load_gather(sd_v, [sidx_v[sl]])
        hist = xs + u_v[sl] * (xe - xs)
        ga = lax.rem(jnp.abs(st * jnp.float32(2.0) + n_v[sl] * st), PI32)
        out_v[sl] = jnp.where(st <= jnp.float32(0.1), ga, hist)
    pltpu.sync_copy(out_v, out_hbm.at[pl.ds(base, _CHUNK)])


@functools.lru_cache(maxsize=1)
def _sc_call_built():
    return pl.kernel(
        _sc_body,
        mesh=plsc.VectorSubcoreMesh(
            core_axis_name="c", subcore_axis_name="s", num_cores=_NC),
        out_type=jax.ShapeDtypeStruct((N_SAMPLES,), jnp.float32),
        compiler_params=pltpu.CompilerParams(needs_layout_passes=False),
        scratch_types=[
            pltpu.VMEM((NUM_BINS,), jnp.float32),
            pltpu.VMEM((64,), jnp.float32),
            pltpu.VMEM((_CHUNK,), jnp.int32),
            pltpu.VMEM((_CHUNK,), jnp.int32),
            pltpu.VMEM((_CHUNK,), jnp.float32),
            pltpu.VMEM((_CHUNK,), jnp.float32),
            pltpu.VMEM((_CHUNK,), jnp.float32),
        ],
    )


def _sc_call(*args):
    return _sc_call_built()(*args)


def kernel(std_idx, X, Y, stddevs):
    size = std_idx.shape
    flat = std_idx.reshape(-1).astype(jnp.int32)
    Y3 = Y.reshape(NROWS, R64, L128)
    bin3, u2, n2 = _tc_call(flat, Y3)
    bin_idx = bin3.reshape(N_SAMPLES)
    u = u2.reshape(N_SAMPLES)
    n = n2.reshape(N_SAMPLES)
    xrow = X[0]
    sd64 = jnp.pad(stddevs, (0, 64 - stddevs.shape[0]))
    out = _sc_call(xrow, sd64, bin_idx, flat, u, n)
    return out.reshape(size)


# speedup vs baseline: 1.6200x; 1.6200x over previous
import functools
import math

import numpy as np
import jax
import jax.numpy as jnp
from jax import lax
from jax.experimental import pallas as pl
from jax.experimental.pallas import tpu as pltpu
from jax.experimental.pallas import tpu_sc as plsc

N_SAMPLES = 8192
NUM_BINS = 8192
NB = NUM_BINS - 1
NROWS = 50
BS = 64
NSTEPS = N_SAMPLES // BS
R64, L128 = 64, 128

K_CAT = (1832780943, 270669613)
K_UNIF = (64467757, 2916123636)
K_NORM = (2465931498, 255383827)

TINY = np.float32(np.finfo(np.float32).tiny)
NORM_LO = np.float32(np.nextafter(np.float32(-1.0), np.float32(0.0)))
NORM_SCALE = np.float32(np.float32(1.0) - NORM_LO)
SQRT2 = np.float32(np.sqrt(2.0))
PI32 = np.float32(math.pi)


def _c32(x):
    return jnp.asarray(np.uint32(x).view(np.int32), dtype=jnp.int32)


def _threefry_xor(key, j):
    k1, k2 = key
    ks0 = _c32(k1)
    ks1 = _c32(k2)
    ks2 = _c32(np.uint32(k1) ^ np.uint32(k2) ^ np.uint32(0x1BD11BDA))

    def rotl(v, d):
        return jnp.bitwise_or(
            jnp.left_shift(v, d), lax.shift_right_logical(v, 32 - d))

    x0 = jnp.full_like(j, 0) + ks0
    x1 = j + ks1
    ks = (ks0, ks1, ks2)
    rot_a = (13, 15, 26, 6)
    rot_b = (17, 29, 16, 24)
    for r in range(5):
        for d in (rot_a if r % 2 == 0 else rot_b):
            x0 = x0 + x1
            x1 = rotl(x1, d)
            x1 = jnp.bitwise_xor(x1, x0)
        x0 = x0 + ks[(r + 1) % 3]
        x1 = x1 + ks[(r + 2) % 3] + (r + 1)
    return jnp.bitwise_xor(x0, x1)


def _bits_to_unit_float(bits):
    mant = jnp.bitwise_or(lax.shift_right_logical(bits, 9), _c32(0x3F800000))
    return lax.bitcast_convert_type(mant, jnp.float32) - jnp.float32(1.0)


def _erf_inv(x):
    w = -jnp.log((jnp.float32(1.0) - x) * (jnp.float32(1.0) + x))
    ws = w - jnp.float32(2.5)
    p_s = jnp.float32(2.81022636e-08)
    for c in (3.43273939e-07, -3.5233877e-06, -4.39150654e-06, 0.00021858087,
              -0.00125372503, -0.00417768164, 0.246640727, 1.50140941):
        p_s = jnp.float32(c) + p_s * ws
    wb = jnp.sqrt(w) - jnp.float32(3.0)
    p_b = jnp.float32(-0.000200214257)
    for c in (0.000100950558, 0.00134934322, -0.00367342844, 0.00573950773,
              -0.0076224613, 0.00943887047, 1.00167406, 2.83297682):
        p_b = jnp.float32(c) + p_b * wb
    return jnp.where(w < jnp.float32(5.0), p_s, p_b) * x


def _tc_body(idx_ref, y_ref, bin_ref, u_ref, n_ref, logit_scr, b2_scr,
             lb_scr):
    pid = pl.program_id(0)

    @pl.when(pid == 0)
    def _init():
        r_i = lax.broadcasted_iota(jnp.int32, (R64, L128), 0)
        l_i = lax.broadcasted_iota(jnp.int32, (R64, L128), 1)
        b2 = r_i * L128 + l_i
        b2_scr[...] = b2
        is_pad = jnp.logical_and(r_i == R64 - 1, l_i == L128 - 1)
        for d in range(NROWS):
            lg = jnp.log(jnp.maximum(y_ref[d], jnp.float32(1e-38)))
            logit_scr[d] = jnp.where(is_pad, jnp.float32(-1e30), lg)
        u_ref[...] = _bits_to_unit_float(_threefry_xor(K_UNIF, b2))
        fn = _bits_to_unit_float(_threefry_xor(K_NORM, b2))
        un = jnp.maximum(NORM_LO, fn * NORM_SCALE + NORM_LO)
        n_ref[...] = SQRT2 * _erf_inv(un)

    base = pid * BS
    for s in range(BS):
        lb_scr[s] = logit_scr[idx_ref[base + s]]
    b3 = jnp.broadcast_to(b2_scr[...], (BS, R64, L128))
    s3 = lax.broadcasted_iota(jnp.int32, (BS, R64, L128), 0)
    j3 = (base + s3) * NB + b3
    f = _bits_to_unit_float(_threefry_xor(K_CAT, j3))
    u = f + TINY
    val = lb_scr[...] - jnp.log(-jnp.log(u))
    m = jnp.max(jnp.max(val, axis=2, keepdims=True), axis=1, keepdims=True)
    cand = jnp.where(val == m, b3, jnp.int32(2**30))
    c = jnp.min(jnp.min(cand, axis=2, keepdims=True), axis=1, keepdims=True)
    bin_ref[0, 0, :] = c.reshape(BS)


def _tc_call(std_idx, Y3, interpret=False):
    grid_spec = pltpu.PrefetchScalarGridSpec(
        num_scalar_prefetch=1,
        grid=(NSTEPS,),
        in_specs=[
            pl.BlockSpec((NROWS, R64, L128), lambda i, *_: (0, 0, 0)),
        ],
        out_specs=[
            pl.BlockSpec((1, 1, BS), lambda i, *_: (i, 0, 0)),
            pl.BlockSpec((R64, L128), lambda i, *_: (0, 0)),
            pl.BlockSpec((R64, L128), lambda i, *_: (0, 0)),
        ],
        scratch_shapes=[pltpu.VMEM((NROWS, R64, L128), jnp.float32),
                        pltpu.VMEM((R64, L128), jnp.int32),
                        pltpu.VMEM((BS, R64, L128), jnp.float32)],
    )
    return pl.pallas_call(
        _tc_body,
        grid_spec=grid_spec,
        out_shape=[
            jax.ShapeDtypeStruct((NSTEPS, 1, BS), jnp.int32),
            jax.ShapeDtypeStruct((R64, L128), jnp.float32),
            jax.ShapeDtypeStruct((R64, L128), jnp.float32),
        ],
        interpret=interpret,
    )(std_idx, Y3)


_NC = 2
_NS = 16
_NW = _NC * _NS
_CHUNK = N_SAMPLES // _NW
_LANES = 16


def _sc_body(xrow_hbm, sd_hbm, bin_hbm, sidx_hbm, u_hbm, n_hbm, out_hbm,
             xrow_v, sd_v, bin_v, sidx_v, u_v, n_v, out_v):
    wid = lax.axis_index("s") * _NC + lax.axis_index("c")
    base = wid * _CHUNK
    pltpu.sync_copy(xrow_hbm, xrow_v)
    pltpu.sync_copy(sd_hbm, sd_v)
    pltpu.sync_copy(bin_hbm.at[pl.ds(base, _CHUNK)], bin_v)
    pltpu.sync_copy(sidx_hbm.at[pl.ds(base, _CHUNK)], sidx_v)
    pltpu.sync_copy(u_hbm.at[pl.ds(base, _CHUNK)], u_v)
    pltpu.sync_copy(n_hbm.at[pl.ds(base, _CHUNK)], n_v)
    for k in range(_CHUNK // _LANES):
        sl = pl.ds(k * _LANES, _LANES)
        bi = bin_v[sl]
        xs = plsc.load_gather(xrow_v, [bi])
        xe = plsc.load_gather(xrow_v, [bi + 1])
        st = plsc.load_gather(sd_v, [sidx_v[sl]])
        hist = xs + u_v[sl] * (xe - xs)
        ga = lax.rem(jnp.abs(st * jnp.float32(2.0) + n_v[sl] * st), PI32)
        out_v[sl] = jnp.where(st <= jnp.float32(0.1), ga, hist)
    pltpu.sync_copy(out_v, out_hbm.at[pl.ds(base, _CHUNK)])


@functools.lru_cache(maxsize=1)
def _sc_call_built():
    return pl.kernel(
        _sc_body,
        mesh=plsc.VectorSubcoreMesh(
            core_axis_name="c", subcore_axis_name="s", num_cores=_NC),
        out_type=jax.ShapeDtypeStruct((N_SAMPLES,), jnp.float32),
        compiler_params=pltpu.CompilerParams(needs_layout_passes=False),
        scratch_types=[
            pltpu.VMEM((NUM_BINS,), jnp.float32),
            pltpu.VMEM((64,), jnp.float32),
            pltpu.VMEM((_CHUNK,), jnp.int32),
            pltpu.VMEM((_CHUNK,), jnp.int32),
            pltpu.VMEM((_CHUNK,), jnp.float32),
            pltpu.VMEM((_CHUNK,), jnp.float32),
            pltpu.VMEM((_CHUNK,), jnp.float32),
        ],
    )


def _sc_call(*args):
    return _sc_call_built()(*args)


def kernel(std_idx, X, Y, stddevs):
    size = std_idx.shape
    flat = std_idx.reshape(-1).astype(jnp.int32)
    Y3 = Y.reshape(NROWS, R64, L128)
    bin3, u2, n2 = _tc_call(flat, Y3)
    bin_idx = bin3.reshape(N_SAMPLES)
    u = u2.reshape(N_SAMPLES)
    n = n2.reshape(N_SAMPLES)
    xrow = X[0]
    sd64 = jnp.pad(stddevs, (0, 64 - stddevs.shape[0]))
    out = _sc_call(xrow, sd64, bin_idx, flat, u, n)
    return out.reshape(size)


# speedup vs baseline: 1.6410x; 1.0129x over previous
import functools
import math

import numpy as np
import jax
import jax.numpy as jnp
from jax import lax
from jax.experimental import pallas as pl
from jax.experimental.pallas import tpu as pltpu
from jax.experimental.pallas import tpu_sc as plsc

N_SAMPLES = 8192
NUM_BINS = 8192
NB = NUM_BINS - 1
NROWS = 50
BS = 128
NSTEPS = N_SAMPLES // BS
R64, L128 = 64, 128

K_CAT = (1832780943, 270669613)
K_UNIF = (64467757, 2916123636)
K_NORM = (2465931498, 255383827)

TINY = np.float32(np.finfo(np.float32).tiny)
NORM_LO = np.float32(np.nextafter(np.float32(-1.0), np.float32(0.0)))
NORM_SCALE = np.float32(np.float32(1.0) - NORM_LO)
SQRT2 = np.float32(np.sqrt(2.0))
PI32 = np.float32(math.pi)


def _c32(x):
    return jnp.asarray(np.uint32(x).view(np.int32), dtype=jnp.int32)


def _threefry_xor(key, j):
    k1, k2 = key
    ks0 = _c32(k1)
    ks1 = _c32(k2)
    ks2 = _c32(np.uint32(k1) ^ np.uint32(k2) ^ np.uint32(0x1BD11BDA))

    def rotl(v, d):
        return jnp.bitwise_or(
            jnp.left_shift(v, d), lax.shift_right_logical(v, 32 - d))

    x0 = jnp.full_like(j, 0) + ks0
    x1 = j + ks1
    ks = (ks0, ks1, ks2)
    rot_a = (13, 15, 26, 6)
    rot_b = (17, 29, 16, 24)
    for r in range(5):
        for d in (rot_a if r % 2 == 0 else rot_b):
            x0 = x0 + x1
            x1 = rotl(x1, d)
            x1 = jnp.bitwise_xor(x1, x0)
        x0 = x0 + ks[(r + 1) % 3]
        x1 = x1 + ks[(r + 2) % 3] + (r + 1)
    return jnp.bitwise_xor(x0, x1)


def _bits_to_unit_float(bits):
    mant = jnp.bitwise_or(lax.shift_right_logical(bits, 9), _c32(0x3F800000))
    return lax.bitcast_convert_type(mant, jnp.float32) - jnp.float32(1.0)


def _erf_inv(x):
    w = -jnp.log((jnp.float32(1.0) - x) * (jnp.float32(1.0) + x))
    ws = w - jnp.float32(2.5)
    p_s = jnp.float32(2.81022636e-08)
    for c in (3.43273939e-07, -3.5233877e-06, -4.39150654e-06, 0.00021858087,
              -0.00125372503, -0.00417768164, 0.246640727, 1.50140941):
        p_s = jnp.float32(c) + p_s * ws
    wb = jnp.sqrt(w) - jnp.float32(3.0)
    p_b = jnp.float32(-0.000200214257)
    for c in (0.000100950558, 0.00134934322, -0.00367342844, 0.00573950773,
              -0.0076224613, 0.00943887047, 1.00167406, 2.83297682):
        p_b = jnp.float32(c) + p_b * wb
    return jnp.where(w < jnp.float32(5.0), p_s, p_b) * x


def _tc_body(idx_ref, y_ref, bin_ref, u_ref, n_ref, logit_scr, b2_scr,
             lb_scr):
    pid = pl.program_id(0)

    @pl.when(pid == 0)
    def _init():
        r_i = lax.broadcasted_iota(jnp.int32, (R64, L128), 0)
        l_i = lax.broadcasted_iota(jnp.int32, (R64, L128), 1)
        b2 = r_i * L128 + l_i
        b2_scr[...] = b2
        is_pad = jnp.logical_and(r_i == R64 - 1, l_i == L128 - 1)
        for d in range(NROWS):
            lg = jnp.log(jnp.maximum(y_ref[d], jnp.float32(1e-38)))
            logit_scr[d] = jnp.where(is_pad, jnp.float32(-1e30), lg)
        u_ref[...] = _bits_to_unit_float(_threefry_xor(K_UNIF, b2))
        fn = _bits_to_unit_float(_threefry_xor(K_NORM, b2))
        un = jnp.maximum(NORM_LO, fn * NORM_SCALE + NORM_LO)
        n_ref[...] = SQRT2 * _erf_inv(un)

    base = pid * BS
    for s in range(BS):
        lb_scr[s] = logit_scr[idx_ref[base + s]]
    b3 = jnp.broadcast_to(b2_scr[...], (BS, R64, L128))
    s3 = lax.broadcasted_iota(jnp.int32, (BS, R64, L128), 0)
    j3 = (base + s3) * NB + b3
    f = _bits_to_unit_float(_threefry_xor(K_CAT, j3))
    u = f + TINY
    val = lb_scr[...] - jnp.log(-jnp.log(u))
    m = jnp.max(jnp.max(val, axis=2, keepdims=True), axis=1, keepdims=True)
    cand = jnp.where(val == m, b3, jnp.int32(2**30))
    c = jnp.min(jnp.min(cand, axis=2, keepdims=True), axis=1, keepdims=True)
    bin_ref[0, 0, :] = c.reshape(BS)


def _tc_call(std_idx, Y3, interpret=False):
    grid_spec = pltpu.PrefetchScalarGridSpec(
        num_scalar_prefetch=1,
        grid=(NSTEPS,),
        in_specs=[
            pl.BlockSpec((NROWS, R64, L128), lambda i, *_: (0, 0, 0)),
        ],
        out_specs=[
            pl.BlockSpec((1, 1, BS), lambda i, *_: (i, 0, 0)),
            pl.BlockSpec((R64, L128), lambda i, *_: (0, 0)),
            pl.BlockSpec((R64, L128), lambda i, *_: (0, 0)),
        ],
        scratch_shapes=[pltpu.VMEM((NROWS, R64, L128), jnp.float32),
                        pltpu.VMEM((R64, L128), jnp.int32),
                        pltpu.VMEM((BS, R64, L128), jnp.float32)],
    )
    return pl.pallas_call(
        _tc_body,
        grid_spec=grid_spec,
        out_shape=[
            jax.ShapeDtypeStruct((NSTEPS, 1, BS), jnp.int32),
            jax.ShapeDtypeStruct((R64, L128), jnp.float32),
            jax.ShapeDtypeStruct((R64, L128), jnp.float32),
        ],
        interpret=interpret,
    )(std_idx, Y3)


_NC = 2
_NS = 16
_NW = _NC * _NS
_CHUNK = N_SAMPLES // _NW
_LANES = 16


def _sc_body(xrow_hbm, sd_hbm, bin_hbm, sidx_hbm, u_hbm, n_hbm, out_hbm,
             xrow_v, sd_v, bin_v, sidx_v, u_v, n_v, out_v):
    wid = lax.axis_index("s") * _NC + lax.axis_index("c")
    base = wid * _CHUNK
    pltpu.sync_copy(xrow_hbm, xrow_v)
    pltpu.sync_copy(sd_hbm, sd_v)
    pltpu.sync_copy(bin_hbm.at[pl.ds(base, _CHUNK)], bin_v)
    pltpu.sync_copy(sidx_hbm.at[pl.ds(base, _CHUNK)], sidx_v)
    pltpu.sync_copy(u_hbm.at[pl.ds(base, _CHUNK)], u_v)
    pltpu.sync_copy(n_hbm.at[pl.ds(base, _CHUNK)], n_v)
    for k in range(_CHUNK // _LANES):
        sl = pl.ds(k * _LANES, _LANES)
        bi = bin_v[sl]
        xs = plsc.load_gather(xrow_v, [bi])
        xe = plsc.load_gather(xrow_v, [bi + 1])
        st = plsc.load_gather(sd_v, [sidx_v[sl]])
        hist = xs + u_v[sl] * (xe - xs)
        ga = lax.rem(jnp.abs(st * jnp.float32(2.0) + n_v[sl] * st), PI32)
        out_v[sl] = jnp.where(st <= jnp.float32(0.1), ga, hist)
    pltpu.sync_copy(out_v, out_hbm.at[pl.ds(base, _CHUNK)])


@functools.lru_cache(maxsize=1)
def _sc_call_built():
    return pl.kernel(
        _sc_body,
        mesh=plsc.VectorSubcoreMesh(
            core_axis_name="c", subcore_axis_name="s", num_cores=_NC),
        out_type=jax.ShapeDtypeStruct((N_SAMPLES,), jnp.float32),
        compiler_params=pltpu.CompilerParams(needs_layout_passes=False),
        scratch_types=[
            pltpu.VMEM((NUM_BINS,), jnp.float32),
            pltpu.VMEM((64,), jnp.float32),
            pltpu.VMEM((_CHUNK,), jnp.int32),
            pltpu.VMEM((_CHUNK,), jnp.int32),
            pltpu.VMEM((_CHUNK,), jnp.float32),
            pltpu.VMEM((_CHUNK,), jnp.float32),
            pltpu.VMEM((_CHUNK,), jnp.float32),
        ],
    )


def _sc_call(*args):
    return _sc_call_built()(*args)


def kernel(std_idx, X, Y, stddevs):
    size = std_idx.shape
    flat = std_idx.reshape(-1).astype(jnp.int32)
    Y3 = Y.reshape(NROWS, R64, L128)
    bin3, u2, n2 = _tc_call(flat, Y3)
    bin_idx = bin3.reshape(N_SAMPLES)
    u = u2.reshape(N_SAMPLES)
    n = n2.reshape(N_SAMPLES)
    xrow = X[0]
    sd64 = jnp.pad(stddevs, (0, 64 - stddevs.shape[0]))
    out = _sc_call(xrow, sd64, bin_idx, flat, u, n)
    return out.reshape(size)
